# baseline (speedup 1.0000x reference)
import functools

import jax
import jax.numpy as jnp
from jax import lax
from jax.experimental import pallas as pl
from jax.experimental.pallas import tpu as pltpu
from jax.experimental.pallas import tpu_sc as plsc

_B, _T, _S, _H, _V = 4, 512, 512, 1024, 32110
_NC, _NS = 2, 16
_NW = _NC * _NS
_ROWS = _B * _T
_RPW = _ROWS // _NW
_NPAIR = _RPW // 2
_NSG = _S // 16


def _sc_body(attn_hbm, ids_hbm, bg_hbm,
             ids_v, cnt_v, vals_v, comb_v, idx_v, accum, sem0, sem1):
    c = lax.axis_index("c")
    s = lax.axis_index("s")
    wid = s * _NC + c
    b = wid // (_NW // _B)
    row0 = wid * _RPW

    pltpu.sync_copy(ids_hbm.at[b], ids_v)
    z16 = jnp.zeros((16,), jnp.float32)

    def zacc(i, cy):
        accum[pl.ds(i * 16, 16)] = z16
        return cy

    lax.fori_loop(0, (_V + 15) // 16, zacc, 0)

    iota16 = lax.iota(jnp.int32, 16)

    def cnt_body(kk, maxk):
        idsk = ids_v[pl.ds(kk * 16, 16)]
        cnt16 = jnp.zeros((16,), jnp.int32)
        for sh in range(1, 16):
            idx = jnp.maximum(iota16 - sh, 0)
            shifted = idsk.at[idx].get(mode="promise_in_bounds")
            eq = (idsk == shifted) & (iota16 >= sh)
            cnt16 = cnt16 + eq.astype(jnp.int32)
        cnt_v[pl.ds(kk * 16, 16)] = cnt16
        return jnp.maximum(maxk, cnt16)

    maxk = lax.fori_loop(0, _NSG, cnt_body, jnp.zeros((16,), jnp.int32))
    nrounds = jnp.max(maxk) + 1

    t0w = row0 - b * _T

    def process_row(t, half):
        pltpu.sync_copy(attn_hbm.at[b, t], vals_v)

        def round_pass(k, c2):
            def sg_body(kk, c3):
                idx16 = ids_v[pl.ds(kk * 16, 16)]
                v16 = vals_v[pl.ds(kk * 16, 16)]
                m16 = cnt_v[pl.ds(kk * 16, 16)] == k
                plsc.addupdate_scatter(accum, [idx16], v16, mask=m16)
                return c3

            return lax.fori_loop(0, _NSG, sg_body, c2)

        lax.fori_loop(0, nrounds, round_pass, 0)

        roff = (t + b * _T) * _V

        def gath_body(kk, c2):
            idx16 = ids_v[pl.ds(kk * 16, 16)]
            comb16 = plsc.load_gather(accum, [idx16])
            comb_v[half, pl.ds(kk * 16, 16)] = comb16
            return c2

        lax.fori_loop(0, _NSG, gath_body, 0)

        def rezero_body(kk, c2):
            idx16 = ids_v[pl.ds(kk * 16, 16)]
            plsc.store_scatter(accum, [idx16], z16)
            return c2

        lax.fori_loop(0, _NSG, rezero_body, 0)

        for j in range(4):
            def idx_body(k, c2):
                idx_v[half, j, pl.ds(k * 16, 16)] = (
                    ids_v[pl.ds(j * 128 + k * 16, 16)] + roff)
                return c2

            lax.fori_loop(0, 8, idx_body, 0)

    def issue_scatters(half, sem):
        for j in range(4):
            pltpu.async_copy(comb_v.at[half, pl.ds(j * 128, 128)],
                             bg_hbm.at[idx_v.at[half, j]], sem)

    def drain(half, sem):
        pltpu.make_async_copy(bg_hbm.at[pl.ds(0, _S)],
                              comb_v.at[half], sem).wait()

    def pair_body(pi, carry):
        t_a = t0w + 2 * pi

        @pl.when(pi >= 1)
        def _():
            drain(0, sem0)

        process_row(t_a, 0)
        issue_scatters(0, sem0)

        @pl.when(pi >= 1)
        def _():
            drain(1, sem1)

        process_row(t_a + 1, 1)
        issue_scatters(1, sem1)
        return carry

    lax.fori_loop(0, _NPAIR, pair_body, 0)
    drain(0, sem0)
    drain(1, sem1)


_sc_scatter = functools.partial(
    pl.kernel,
    mesh=plsc.VectorSubcoreMesh(core_axis_name="c", subcore_axis_name="s",
                                num_cores=_NC, num_subcores=_NS),
    compiler_params=pltpu.CompilerParams(needs_layout_passes=False),
    scratch_types=[
        pltpu.VMEM((_S,), jnp.int32),
        pltpu.VMEM((_S,), jnp.int32),
        pltpu.VMEM((_S,), jnp.float32),
        pltpu.VMEM((2, _S), jnp.float32),
        pltpu.VMEM((2, 4, 128), jnp.int32),
        pltpu.VMEM((_V,), jnp.float32),
        pltpu.SemaphoreType.DMA,
        pltpu.SemaphoreType.DMA,
    ],
)(_sc_body)


_ZROWS = 64


def _zfill_body(out_ref):
    out_ref[...] = jnp.zeros((_ZROWS, _V), jnp.float32)


_zfill = pl.pallas_call(
    _zfill_body,
    grid=(_ROWS // _ZROWS,),
    out_specs=pl.BlockSpec((_ZROWS, _V), lambda i: (i, 0)),
    out_shape=jax.ShapeDtypeStruct((_ROWS, _V), jnp.float32),
)


def _pgen_body(dec_ref, seq_ref, w1_ref, w2_ref, b_ref, out_ref):
    d = dec_ref[...]
    q = seq_ref[...]
    acc = (jnp.sum(d * w1_ref[0][None, None, :], axis=2)
           + jnp.sum(q * w2_ref[0][None, None, :], axis=2)
           + b_ref[0, 0])
    out_ref[...] = jax.nn.sigmoid(acc)


_pgen = pl.pallas_call(
    _pgen_body,
    out_shape=jax.ShapeDtypeStruct((_B, _T), jnp.float32),
)


def kernel(decoder_input_embeds, sequence_output, cross_attentions,
           input_ids_to_copy, W, b):
    w1 = W[:_H, 0].reshape(1, _H)
    w2 = W[_H:, 0].reshape(1, _H)
    p_gen = _pgen(decoder_input_embeds, sequence_output, w1, w2,
                  b.reshape(1, 1)).reshape(_B, _T, 1)
    bg = _zfill().reshape(_ROWS * _V)
    ref = jax.new_ref(bg)
    _sc_scatter(cross_attentions, input_ids_to_copy, ref)
    return (p_gen, ref[...].reshape(_B, _T, _V))


# speedup vs baseline: 11.3284x; 11.3284x over previous
import jax
import jax.numpy as jnp
from jax import lax
from jax.experimental import pallas as pl
from jax.experimental.pallas import tpu as pltpu

_B, _T, _S, _H, _V = 4, 512, 512, 1024, 32110
_VT = 512
_NJ = (_V + _VT - 1) // _VT


def _logits_body(ids_ref, attn_ref, out_ref):
    j = pl.program_id(1)
    ids = ids_ref[0, 0, :]
    iota_v = lax.broadcasted_iota(jnp.int32, (_S, _VT), 1) + j * _VT
    onehot = (ids[:, None] == iota_v).astype(jnp.bfloat16)
    a = attn_ref[0].astype(jnp.bfloat16)
    out_ref[0] = jnp.dot(a, onehot, preferred_element_type=jnp.float32)


_logits = pl.pallas_call(
    _logits_body,
    grid=(_B, _NJ),
    in_specs=[
        pl.BlockSpec((1, 1, _S), lambda b, j: (b, 0, 0)),
        pl.BlockSpec((1, _T, _S), lambda b, j: (b, 0, 0)),
    ],
    out_specs=pl.BlockSpec((1, _T, _VT), lambda b, j: (b, 0, j)),
    out_shape=jax.ShapeDtypeStruct((_B, _T, _V), jnp.float32),
    compiler_params=pltpu.CompilerParams(
        dimension_semantics=("parallel", "parallel")),
)


def _pgen_body(dec_ref, seq_ref, w1_ref, w2_ref, b_ref, out_ref):
    d = dec_ref[...]
    q = seq_ref[...]
    acc = (jnp.sum(d * w1_ref[0][None, None, :], axis=2)
           + jnp.sum(q * w2_ref[0][None, None, :], axis=2)
           + b_ref[0, 0])
    out_ref[...] = jax.nn.sigmoid(acc)


_pgen = pl.pallas_call(
    _pgen_body,
    out_shape=jax.ShapeDtypeStruct((_B, _T), jnp.float32),
)


def kernel(decoder_input_embeds, sequence_output, cross_attentions,
           input_ids_to_copy, W, b):
    w1 = W[:_H, 0].reshape(1, _H)
    w2 = W[_H:, 0].reshape(1, _H)
    p_gen = _pgen(decoder_input_embeds, sequence_output, w1, w2,
                  b.reshape(1, 1)).reshape(_B, _T, 1)
    logits = _logits(input_ids_to_copy.reshape(_B, 1, _S), cross_attentions)
    return (p_gen, logits)


# speedup vs baseline: 42.1013x; 3.7164x over previous
import jax
import jax.numpy as jnp
from jax import lax
from jax.experimental import pallas as pl
from jax.experimental.pallas import tpu as pltpu

_B, _T, _S, _H, _V = 4, 512, 512, 1024, 32110
_VT = 512
_NJ = (_V + _VT - 1) // _VT


def _logits_body(ids_ref, attn_t_ref, out_ref):
    j = pl.program_id(0)
    iota_v = lax.broadcasted_iota(jnp.int32, (_VT, _S), 0) + j * _VT
    for b in range(_B):
        ids_b = ids_ref[b, 0, :]
        onehot_t = (iota_v == ids_b[None, :]).astype(jnp.bfloat16)
        a_b = attn_t_ref[b].astype(jnp.bfloat16)
        out_ref[:, b, :] = jnp.dot(onehot_t, a_b,
                                   preferred_element_type=jnp.float32)


_logits_t = pl.pallas_call(
    _logits_body,
    grid=(_NJ,),
    in_specs=[
        pl.BlockSpec((_B, 1, _S), lambda j: (0, 0, 0)),
        pl.BlockSpec((_B, _S, _T), lambda j: (0, 0, 0)),
    ],
    out_specs=pl.BlockSpec((_VT, _B, _T), lambda j: (j, 0, 0)),
    out_shape=jax.ShapeDtypeStruct((_V, _B, _T), jnp.float32),
    compiler_params=pltpu.CompilerParams(
        dimension_semantics=("parallel",)),
)


def _pgen_body(dec_ref, seq_ref, w1_ref, w2_ref, b_ref, out_ref):
    d = dec_ref[...]
    q = seq_ref[...]
    acc = (jnp.sum(d * w1_ref[0][None, None, :], axis=2)
           + jnp.sum(q * w2_ref[0][None, None, :], axis=2)
           + b_ref[0, 0])
    out_ref[...] = jax.nn.sigmoid(acc)


_pgen = pl.pallas_call(
    _pgen_body,
    out_shape=jax.ShapeDtypeStruct((_B, _T), jnp.float32),
)


def kernel(decoder_input_embeds, sequence_output, cross_attentions,
           input_ids_to_copy, W, b):
    w1 = W[:_H, 0].reshape(1, _H)
    w2 = W[_H:, 0].reshape(1, _H)
    p_gen = _pgen(decoder_input_embeds, sequence_output, w1, w2,
                  b.reshape(1, 1)).reshape(_B, _T, 1)
    attn_t = cross_attentions.transpose(0, 2, 1)
    out_t = _logits_t(input_ids_to_copy.reshape(_B, 1, _S), attn_t)
    logits = out_t.transpose(1, 2, 0)
    return (p_gen, logits)
